# baseline (speedup 1.0000x reference)
import functools

import jax
import jax.numpy as jnp
from jax import lax
from jax.experimental import pallas as pl
from jax.experimental.pallas import tpu as pltpu
from jax.experimental.pallas import tpu_sc as plsc

D = 128
LANES = 16
NC, NS = 2, 16
NW = NC * NS
E_TOTAL = 320000
E_PER_W = E_TOTAL // NW
CHUNK = 16
NCHUNK = E_PER_W // CHUNK

_GATHER_DN = lax.GatherDimensionNumbers(
    offset_dims=(), collapsed_slice_dims=(0,), start_index_map=(0,))


def _lane_perm(x, idx):
    return lax.gather(x, idx[:, None], _GATHER_DN, slice_sizes=(1,),
                      mode=lax.GatherScatterMode.PROMISE_IN_BOUNDS)


def _dot_decoder_sc(z, cidx):
    mesh = plsc.VectorSubcoreMesh(core_axis_name="c", subcore_axis_name="s")

    @functools.partial(
        pl.kernel,
        mesh=mesh,
        out_type=jax.ShapeDtypeStruct((E_TOTAL,), jnp.float32),
        scratch_types=[
            pltpu.VMEM((2 * E_PER_W,), jnp.int32),
            pltpu.VMEM((2 * CHUNK, D), jnp.float32),
            pltpu.VMEM((2 * CHUNK, D), jnp.float32),
            pltpu.VMEM_SHARED((10000, D), jnp.float32),
            pltpu.VMEM((E_PER_W,), jnp.float32),
            pltpu.SemaphoreType.DMA,
            pltpu.SemaphoreType.DMA,
        ],
    )
    def k(z_hbm, cidx_hbm, out_hbm,
          cidx, rows_a, rows_b, zsh, outv, sem_a, sem_b):
        sid = lax.axis_index("s")
        wid = sid * NC + lax.axis_index("c")
        base = wid * E_PER_W

        @pl.when(sid == 0)
        def _():
            pltpu.sync_copy(z_hbm, zsh)

        pltpu.sync_copy(cidx_hbm.at[pl.ds(2 * base, 2 * E_PER_W)], cidx)
        plsc.subcore_barrier()

        lane = lax.iota(jnp.int32, 16)

        def fire(j, rows, sem):
            pltpu.async_copy(zsh.at[cidx.at[pl.ds(j * 2 * CHUNK, 2 * CHUNK)]],
                             rows, sem)

        def drain(rows, sem):
            pltpu.make_async_copy(zsh.at[cidx.at[pl.ds(0, 2 * CHUNK)]],
                                  rows, sem).wait()

        def compute(j, rows):
            c0 = j * CHUNK
            out16 = jnp.zeros((LANES,), jnp.float32)
            for i in range(LANES):
                acc = jnp.zeros((LANES,), jnp.float32)
                for f in range(D // LANES):
                    acc = acc + (rows[i, pl.ds(f * LANES, LANES)]
                                 * rows[LANES + i, pl.ds(f * LANES, LANES)])
                for sh in (8, 4, 2, 1):
                    acc = acc + _lane_perm(acc, lane ^ sh)
                out16 = jnp.where(lane == i, acc, out16)
            outv[pl.ds(c0, LANES)] = out16

        fire(0, rows_a, sem_a)

        def pair_body(p, _):
            g = p * 2
            fire(g + 1, rows_b, sem_b)
            drain(rows_a, sem_a)
            compute(g, rows_a)
            fire(g + 2, rows_a, sem_a)
            drain(rows_b, sem_b)
            compute(g + 1, rows_b)
            return ()

        lax.fori_loop(0, (NCHUNK - 1) // 2, pair_body, (), unroll=False)

        drain(rows_a, sem_a)
        compute(NCHUNK - 1, rows_a)

        pltpu.sync_copy(outv, out_hbm.at[pl.ds(base, E_PER_W)])

    return k(z, cidx)


def kernel(z, edge_index):
    idx = edge_index.astype(jnp.int32).reshape(2, NW, NCHUNK, CHUNK)
    cidx = jnp.transpose(idx, (1, 2, 0, 3)).reshape(-1)
    return _dot_decoder_sc(z, cidx)


# speedup vs baseline: 1.3838x; 1.3838x over previous
import functools

import jax
import jax.numpy as jnp
from jax import lax
from jax.experimental import pallas as pl
from jax.experimental.pallas import tpu as pltpu
from jax.experimental.pallas import tpu_sc as plsc

D = 128
LANES = 16
NC, NS = 2, 16
NW = NC * NS
E_TOTAL = 320000
E_PER_W = E_TOTAL // NW
CHUNK = 16
NCHUNK = E_PER_W // CHUNK

_GATHER_DN = lax.GatherDimensionNumbers(
    offset_dims=(), collapsed_slice_dims=(0,), start_index_map=(0,))


def _lane_perm(x, idx):
    return lax.gather(x, idx[:, None], _GATHER_DN, slice_sizes=(1,),
                      mode=lax.GatherScatterMode.PROMISE_IN_BOUNDS)


def _dot_decoder_sc(z, src, dst):
    mesh = plsc.VectorSubcoreMesh(core_axis_name="c", subcore_axis_name="s")

    @functools.partial(
        pl.kernel,
        mesh=mesh,
        out_type=jax.ShapeDtypeStruct((E_TOTAL,), jnp.float32),
        scratch_types=[
            pltpu.VMEM((E_PER_W,), jnp.int32),
            pltpu.VMEM((E_PER_W,), jnp.int32),
            pltpu.VMEM((CHUNK, D), jnp.float32),
            pltpu.VMEM((CHUNK, D), jnp.float32),
            pltpu.VMEM((CHUNK, D), jnp.float32),
            pltpu.VMEM((CHUNK, D), jnp.float32),
            pltpu.VMEM_SHARED((10000, D), jnp.float32),
            pltpu.VMEM((E_PER_W,), jnp.float32),
            pltpu.SemaphoreType.DMA,
            pltpu.SemaphoreType.DMA,
            pltpu.SemaphoreType.DMA,
            pltpu.SemaphoreType.DMA,
        ],
    )
    def k(z_hbm, src_hbm, dst_hbm, out_hbm,
          sidx, didx, srows_a, drows_a, srows_b, drows_b, zsh, outv,
          sem_sa, sem_da, sem_sb, sem_db):
        sid = lax.axis_index("s")
        wid = sid * NC + lax.axis_index("c")
        base = wid * E_PER_W

        @pl.when(sid == 0)
        def _():
            pltpu.sync_copy(z_hbm, zsh)

        pltpu.sync_copy(src_hbm.at[pl.ds(base, E_PER_W)], sidx)
        pltpu.sync_copy(dst_hbm.at[pl.ds(base, E_PER_W)], didx)
        plsc.subcore_barrier()

        lane = lax.iota(jnp.int32, 16)

        def fire(j, srows, drows, sem_s, sem_d):
            c0 = j * CHUNK
            pltpu.async_copy(zsh.at[sidx.at[pl.ds(c0, CHUNK)]], srows, sem_s)
            pltpu.async_copy(zsh.at[didx.at[pl.ds(c0, CHUNK)]], drows, sem_d)

        def drain(srows, drows, sem_s, sem_d):
            pltpu.make_async_copy(zsh.at[sidx.at[pl.ds(0, CHUNK)]],
                                  srows, sem_s).wait()
            pltpu.make_async_copy(zsh.at[didx.at[pl.ds(0, CHUNK)]],
                                  drows, sem_d).wait()

        def compute(j, srows, drows):
            c0 = j * CHUNK
            out16 = jnp.zeros((LANES,), jnp.float32)
            for i in range(LANES):
                acc = jnp.zeros((LANES,), jnp.float32)
                for f in range(D // LANES):
                    acc = acc + (srows[i, pl.ds(f * LANES, LANES)]
                                 * drows[i, pl.ds(f * LANES, LANES)])
                for sh in (8, 4, 2, 1):
                    acc = acc + _lane_perm(acc, lane ^ sh)
                out16 = jnp.where(lane == i, acc, out16)
            outv[pl.ds(c0, LANES)] = out16

        fire(0, srows_a, drows_a, sem_sa, sem_da)

        def pair_body(p, _):
            g = p * 2
            fire(g + 1, srows_b, drows_b, sem_sb, sem_db)
            drain(srows_a, drows_a, sem_sa, sem_da)
            compute(g, srows_a, drows_a)
            fire(g + 2, srows_a, drows_a, sem_sa, sem_da)
            drain(srows_b, drows_b, sem_sb, sem_db)
            compute(g + 1, srows_b, drows_b)
            return ()

        lax.fori_loop(0, (NCHUNK - 1) // 2, pair_body, (), unroll=False)

        drain(srows_a, drows_a, sem_sa, sem_da)
        compute(NCHUNK - 1, srows_a, drows_a)

        pltpu.sync_copy(outv, out_hbm.at[pl.ds(base, E_PER_W)])

    return k(z, src, dst)


def kernel(z, edge_index):
    src = edge_index[0].astype(jnp.int32)
    dst = edge_index[1].astype(jnp.int32)
    return _dot_decoder_sc(z, src, dst)
